# baseline (speedup 1.0000x reference)
import dataclasses
import functools

import jax
import jax.numpy as jnp
from jax import lax
from jax.experimental import pallas as pl
from jax.experimental.pallas import tpu as pltpu
from jax.experimental.pallas import tpu_sc as plsc

N_NODES = 10000
N_EDGES = 320000
HID = 128
NW = 32
LANES = 16
NPAD = 10240
NPT = NPAD // 16
ZB = 8
EPAD = 327680
EPW = EPAD // NW

CBA = 128
NCA = EPW // CBA
NCI = 16
NSUP = NCA // NCI

BN = 1024
BE = 4096


def _zero_fill(buf, rows, width):
    @pl.loop(0, rows)
    def _r(r):
        for j in range(width // LANES):
            buf[r, pl.ds(j * LANES, LANES)] = jnp.zeros((LANES,), jnp.float32)


def _mesh():
    return plsc.VectorSubcoreMesh(core_axis_name="c", subcore_axis_name="s")


def _wid():
    return lax.axis_index("s") * 2 + lax.axis_index("c")


def _deg_body(dst_hbm, out_hbm, idx2, cnt_v, sem):
    wid = _wid()

    @pl.loop(0, NPAD // LANES)
    def _z(r):
        cnt_v[pl.ds(r * LANES, LANES)] = jnp.zeros((LANES,), jnp.float32)

    pltpu.sync_copy(dst_hbm.at[wid], idx2)
    ones = jnp.ones((LANES,), jnp.float32)

    @pl.loop(0, NCA)
    def _chunk(i):
        @pl.loop(0, CBA // LANES)
        def _g(g):
            idx = idx2[i, pl.ds(g * LANES, LANES)]
            plsc.addupdate_scatter(cnt_v, [idx], ones)

    pltpu.sync_copy(cnt_v, out_hbm.at[wid])


@jax.jit
def _deg_kernel(dst3):
    cp = pltpu.CompilerParams()
    if "needs_layout_passes" in pltpu.CompilerParams.__dataclass_fields__:
        cp = dataclasses.replace(cp, needs_layout_passes=False)
    k = pl.kernel(
        _deg_body,
        out_type=jax.ShapeDtypeStruct((NW, NPAD), jnp.float32),
        mesh=_mesh(),
        compiler_params=cp,
        scratch_types=[
            pltpu.VMEM((NCA, CBA), jnp.int32),
            pltpu.VMEM((NPAD,), jnp.float32),
            pltpu.SemaphoreType.DMA,
        ],
    )
    return k(dst3)


def _agg_body(u_hbm, src_hbm, dst_hbm, out_hbm,
              sidxs, didxs, rows_v, zero_v, acc_sh, sem):
    cid = lax.axis_index("c")
    sid = lax.axis_index("s")
    wid = sid * 2 + cid

    _zero_fill(zero_v, ZB, HID)
    for t in range(NPT // ZB):
        pltpu.sync_copy(zero_v, acc_sh.at[pl.ds(sid * NPT + t * ZB, ZB)])
    plsc.subcore_barrier()

    @pl.loop(0, NSUP)
    def _sup(sc):
        pltpu.sync_copy(src_hbm.at[wid, pl.ds(sc * NCI, NCI)], sidxs)
        pltpu.sync_copy(dst_hbm.at[wid, pl.ds(sc * NCI, NCI)], didxs)

        @pl.loop(0, NCI)
        def _chunk(i):
            pltpu.async_copy(u_hbm.at[sidxs.at[i]], rows_v, sem).wait()
            pltpu.sync_copy(rows_v, acc_sh.at[didxs.at[i]], add=True)

    plsc.subcore_barrier()
    row0 = cid * NPAD + sid * NPT
    pltpu.sync_copy(acc_sh.at[pl.ds(sid * NPT, NPT)],
                    out_hbm.at[pl.ds(row0, NPT)])


@jax.jit
def _agg_kernel(u, src3, dst3):
    k = pl.kernel(
        _agg_body,
        out_type=jax.ShapeDtypeStruct((2 * NPAD, HID), jnp.float32),
        mesh=_mesh(),
        scratch_types=[
            pltpu.VMEM((NCI, CBA), jnp.int32),
            pltpu.VMEM((NCI, CBA), jnp.int32),
            pltpu.VMEM((CBA, HID), jnp.float32),
            pltpu.VMEM((ZB, HID), jnp.float32),
            pltpu.VMEM_SHARED((NPAD, HID), jnp.float32),
            pltpu.SemaphoreType.DMA,
        ],
    )
    return k(u, src3, dst3)


def _edge_stage_body(a_hbm, b_hbm, src_hbm, dst_hbm, c_hbm, w_hbm, out_hbm,
                     sidxs, didxs, rows_a, rows_b, c_v, w_v, o_v, sem):
    wid = _wid()
    base = wid * EPW
    pltpu.sync_copy(w_hbm, w_v)

    @pl.loop(0, NSUP)
    def _sup(sc):
        pltpu.sync_copy(src_hbm.at[wid, pl.ds(sc * NCI, NCI)], sidxs)
        pltpu.sync_copy(dst_hbm.at[wid, pl.ds(sc * NCI, NCI)], didxs)

        @pl.loop(0, NCI)
        def _chunk(i):
            off = base + sc * NCI * CBA + i * CBA
            ca = pltpu.async_copy(a_hbm.at[sidxs.at[i]], rows_a, sem)
            cb = pltpu.async_copy(b_hbm.at[didxs.at[i]], rows_b, sem)
            cc = pltpu.async_copy(c_hbm.at[pl.ds(off, CBA)], c_v, sem)
            ca.wait()
            cb.wait()
            cc.wait()

            @pl.loop(0, CBA)
            def _edge(e):
                acc = jnp.zeros((LANES,), jnp.float32)
                for j in range(HID // LANES):
                    sl = pl.ds(j * LANES, LANES)
                    g = rows_a[e, sl] + rows_b[e, sl] + c_v[e, sl]
                    g = jnp.maximum(g, 0.0)
                    acc = acc + g * w_v[sl]
                o_v[e, :] = acc

            pltpu.sync_copy(o_v, out_hbm.at[pl.ds(off, CBA)])


@jax.jit
def _edge_stage(A, B, src3, dst3, c, w):
    k = pl.kernel(
        _edge_stage_body,
        out_type=jax.ShapeDtypeStruct((EPAD, LANES), jnp.float32),
        mesh=_mesh(),
        scratch_types=[
            pltpu.VMEM((NCI, CBA), jnp.int32),
            pltpu.VMEM((NCI, CBA), jnp.int32),
            pltpu.VMEM((CBA, HID), jnp.float32),
            pltpu.VMEM((CBA, HID), jnp.float32),
            pltpu.VMEM((CBA, HID), jnp.float32),
            pltpu.VMEM((HID,), jnp.float32),
            pltpu.VMEM((CBA, LANES), jnp.float32),
            pltpu.SemaphoreType.DMA,
        ],
    )
    return k(A, B, src3, dst3, c, w)


def _dis_of(degT_blk):
    deg = jnp.sum(degT_blk, axis=1, keepdims=True) + 1.0
    return jax.lax.rsqrt(deg)


def _s1_body(degT, x, W, u):
    u[...] = jnp.dot(x[...], W[...],
                     preferred_element_type=jnp.float32) * _dis_of(degT[...])


def _s2_body(degT, a0, a1, up, b, W, u2):
    dis = _dis_of(degT[...])
    h = jnp.maximum((a0[...] + a1[...] + up[...]) * dis + b[...], 0.0)
    u2[...] = jnp.dot(h, W[...], preferred_element_type=jnp.float32) * dis


def _s3_body(degT, a0, a1, up, b, Wa, Wb, A, B):
    dis = _dis_of(degT[...])
    h = (a0[...] + a1[...] + up[...]) * dis + b[...]
    A[...] = jnp.dot(h, Wa[...], preferred_element_type=jnp.float32)
    B[...] = jnp.dot(h, Wb[...], preferred_element_type=jnp.float32)


def _s4_body(e, W, b, c):
    c[...] = jnp.dot(e[...], W[...],
                     preferred_element_type=jnp.float32) + b[...]


def _s5_body(p, b, o):
    o[...] = jnp.sum(p[...], axis=1, keepdims=True) + b[...]


def _node_spec(w=HID):
    return pl.BlockSpec((BN, w), lambda i: (i, 0))


def _full(shape):
    return pl.BlockSpec(shape, lambda i: tuple(0 for _ in shape))


_GRID_N = NPAD // BN


@jax.jit
def _s1(degT, x, W):
    return pl.pallas_call(
        _s1_body,
        grid=(_GRID_N,),
        in_specs=[_node_spec(NW), _node_spec(), _full((HID, HID))],
        out_specs=_node_spec(),
        out_shape=jax.ShapeDtypeStruct((NPAD, HID), jnp.float32),
    )(degT, x, W)


@jax.jit
def _s2(degT, aggp, up, b, W):
    return pl.pallas_call(
        _s2_body,
        grid=(_GRID_N,),
        in_specs=[_node_spec(NW),
                  _node_spec(),
                  pl.BlockSpec((BN, HID), lambda i: (i + _GRID_N, 0)),
                  _node_spec(), _full((1, HID)), _full((HID, HID))],
        out_specs=_node_spec(),
        out_shape=jax.ShapeDtypeStruct((NPAD, HID), jnp.float32),
    )(degT, aggp, aggp, up, b, W)


@jax.jit
def _s3(degT, aggp, up, b, Wa, Wb):
    return pl.pallas_call(
        _s3_body,
        grid=(_GRID_N,),
        in_specs=[_node_spec(NW),
                  _node_spec(),
                  pl.BlockSpec((BN, HID), lambda i: (i + _GRID_N, 0)),
                  _node_spec(), _full((1, HID)),
                  _full((HID, HID)), _full((HID, HID))],
        out_specs=[_node_spec(), _node_spec()],
        out_shape=[jax.ShapeDtypeStruct((NPAD, HID), jnp.float32),
                   jax.ShapeDtypeStruct((NPAD, HID), jnp.float32)],
    )(degT, aggp, aggp, up, b, Wa, Wb)


@jax.jit
def _s4(e, W, b):
    ef = e.shape[1]
    return pl.pallas_call(
        _s4_body,
        grid=(EPAD // BE,),
        in_specs=[pl.BlockSpec((BE, ef), lambda i: (i, 0)),
                  _full((ef, HID)), _full((1, HID))],
        out_specs=pl.BlockSpec((BE, HID), lambda i: (i, 0)),
        out_shape=jax.ShapeDtypeStruct((EPAD, HID), jnp.float32),
    )(e, W, b)


@jax.jit
def _s5(part, b):
    return pl.pallas_call(
        _s5_body,
        grid=(EPAD // BE,),
        in_specs=[pl.BlockSpec((BE, LANES), lambda i: (i, 0)),
                  _full((1, 1))],
        out_specs=pl.BlockSpec((BE, 1), lambda i: (i, 0)),
        out_shape=jax.ShapeDtypeStruct((EPAD, 1), jnp.float32),
    )(part, b)


def kernel(x, eIndex, eAttributes, W1, b1, W2, b2, Wm1, bm1, Wm2, bm2):
    src = eIndex[0].astype(jnp.int32)
    dst = eIndex[1].astype(jnp.int32)
    H = W1.shape[1]

    pad = EPAD - N_EDGES
    srcp = jnp.concatenate([src, jnp.full((pad,), N_NODES, jnp.int32)])
    dstp = jnp.concatenate([dst, jnp.full((pad,), N_NODES, jnp.int32)])
    src3 = srcp.reshape(NW, NCA, CBA)
    dst3 = dstp.reshape(NW, NCA, CBA)

    def pad_nodes(m):
        return jnp.concatenate(
            [m, jnp.zeros((NPAD - m.shape[0], m.shape[1]), m.dtype)])

    xp = pad_nodes(x)
    degp = _deg_kernel(dst3)
    degT = degp.T

    u = _s1(degT, xp, W1)
    aggp = _agg_kernel(u, src3, dst3)
    u2 = _s2(degT, aggp, u, b1.reshape(1, H), W2)
    aggp2 = _agg_kernel(u2, src3, dst3)
    A, B = _s3(degT, aggp2, u2, b2.reshape(1, H), Wm1[:H], Wm1[H:2 * H])

    eAp = jnp.concatenate(
        [eAttributes, jnp.zeros((pad, eAttributes.shape[1]), jnp.float32)])
    c = _s4(eAp, Wm1[2 * H:], bm1.reshape(1, H))

    part = _edge_stage(A, B, src3, dst3, c, Wm2[:, 0])
    out = _s5(part, bm2.reshape(1, 1))
    return out[:N_EDGES]


# speedup vs baseline: 1.5480x; 1.5480x over previous
import dataclasses
import functools

import jax
import jax.numpy as jnp
from jax import lax
from jax.experimental import pallas as pl
from jax.experimental.pallas import tpu as pltpu
from jax.experimental.pallas import tpu_sc as plsc

N_NODES = 10000
N_EDGES = 320000
HID = 128
NW = 32
LANES = 16
NPAD = 10240
NPT = NPAD // 16
ZB = 8
EPW = N_EDGES // NW
CB = 80
NCHUNK = EPW // CB

BN = 1024
BE = 3200


def _zero_fill(buf, rows, width):
    @pl.loop(0, rows)
    def _r(r):
        for j in range(width // LANES):
            buf[r, pl.ds(j * LANES, LANES)] = jnp.zeros((LANES,), jnp.float32)


def _mesh():
    return plsc.VectorSubcoreMesh(core_axis_name="c", subcore_axis_name="s")


def _wid():
    return lax.axis_index("s") * 2 + lax.axis_index("c")


def _deg_body(dst_hbm, out_hbm, didx, cnt_v, sem):
    wid = _wid()

    @pl.loop(0, NPAD // LANES)
    def _z(r):
        cnt_v[pl.ds(r * LANES, LANES)] = jnp.zeros((LANES,), jnp.float32)

    base = wid * EPW
    ones = jnp.ones((LANES,), jnp.float32)

    @pl.loop(0, NCHUNK)
    def _chunk(i):
        pltpu.sync_copy(dst_hbm.at[pl.ds(base + i * CB, CB)], didx)

        @pl.loop(0, CB // LANES)
        def _g(g):
            idx = didx[pl.ds(g * LANES, LANES)]
            plsc.addupdate_scatter(cnt_v, [idx], ones)

    pltpu.sync_copy(cnt_v, out_hbm.at[wid])


@jax.jit
def _deg_kernel(dst):
    cp = pltpu.CompilerParams()
    if "needs_layout_passes" in pltpu.CompilerParams.__dataclass_fields__:
        cp = dataclasses.replace(cp, needs_layout_passes=False)
    k = pl.kernel(
        _deg_body,
        out_type=jax.ShapeDtypeStruct((NW, NPAD), jnp.float32),
        mesh=_mesh(),
        compiler_params=cp,
        scratch_types=[
            pltpu.VMEM((CB,), jnp.int32),
            pltpu.VMEM((NPAD,), jnp.float32),
            pltpu.SemaphoreType.DMA,
        ],
    )
    return k(dst)


def _agg_body(u_hbm, src_hbm, dst_hbm, out_hbm,
              sidx, didx, rows_v, zero_v, acc_sh, sem):
    cid = lax.axis_index("c")
    sid = lax.axis_index("s")
    wid = sid * 2 + cid

    _zero_fill(zero_v, ZB, HID)
    for t in range(NPT // ZB):
        pltpu.sync_copy(zero_v, acc_sh.at[pl.ds(sid * NPT + t * ZB, ZB)])
    plsc.subcore_barrier()

    base = wid * EPW

    @pl.loop(0, NCHUNK)
    def _chunk(i):
        off = base + i * CB
        pltpu.sync_copy(src_hbm.at[pl.ds(off, CB)], sidx)
        pltpu.sync_copy(dst_hbm.at[pl.ds(off, CB)], didx)
        pltpu.async_copy(u_hbm.at[sidx], rows_v, sem).wait()
        pltpu.sync_copy(rows_v, acc_sh.at[didx], add=True)

    plsc.subcore_barrier()
    row0 = cid * NPAD + sid * NPT
    pltpu.sync_copy(acc_sh.at[pl.ds(sid * NPT, NPT)],
                    out_hbm.at[pl.ds(row0, NPT)])


@jax.jit
def _agg_kernel(u, src, dst):
    k = pl.kernel(
        _agg_body,
        out_type=jax.ShapeDtypeStruct((2 * NPAD, HID), jnp.float32),
        mesh=_mesh(),
        scratch_types=[
            pltpu.VMEM((CB,), jnp.int32),
            pltpu.VMEM((CB,), jnp.int32),
            pltpu.VMEM((CB, HID), jnp.float32),
            pltpu.VMEM((ZB, HID), jnp.float32),
            pltpu.VMEM_SHARED((NPAD, HID), jnp.float32),
            pltpu.SemaphoreType.DMA,
        ],
    )
    return k(u, src, dst)


def _edge_stage_body(a_hbm, b_hbm, src_hbm, dst_hbm, c_hbm, w_hbm, out_hbm,
                     sidx, didx, rows_a, rows_b, c_v, w_v, o_v, sem):
    wid = _wid()
    base = wid * EPW
    pltpu.sync_copy(w_hbm, w_v)

    @pl.loop(0, NCHUNK)
    def _chunk(i):
        off = base + i * CB
        pltpu.sync_copy(src_hbm.at[pl.ds(off, CB)], sidx)
        pltpu.sync_copy(dst_hbm.at[pl.ds(off, CB)], didx)
        ca = pltpu.async_copy(a_hbm.at[sidx], rows_a, sem)
        cb = pltpu.async_copy(b_hbm.at[didx], rows_b, sem)
        cc = pltpu.async_copy(c_hbm.at[pl.ds(off, CB)], c_v, sem)
        ca.wait()
        cb.wait()
        cc.wait()

        @pl.loop(0, CB)
        def _edge(e):
            acc = jnp.zeros((LANES,), jnp.float32)
            for j in range(HID // LANES):
                sl = pl.ds(j * LANES, LANES)
                g = rows_a[e, sl] + rows_b[e, sl] + c_v[e, sl]
                g = jnp.maximum(g, 0.0)
                acc = acc + g * w_v[sl]
            o_v[e, :] = acc

        pltpu.sync_copy(o_v, out_hbm.at[pl.ds(off, CB)])


@jax.jit
def _edge_stage(A, B, src, dst, c, w):
    k = pl.kernel(
        _edge_stage_body,
        out_type=jax.ShapeDtypeStruct((N_EDGES, LANES), jnp.float32),
        mesh=_mesh(),
        scratch_types=[
            pltpu.VMEM((CB,), jnp.int32),
            pltpu.VMEM((CB,), jnp.int32),
            pltpu.VMEM((CB, HID), jnp.float32),
            pltpu.VMEM((CB, HID), jnp.float32),
            pltpu.VMEM((CB, HID), jnp.float32),
            pltpu.VMEM((HID,), jnp.float32),
            pltpu.VMEM((CB, LANES), jnp.float32),
            pltpu.SemaphoreType.DMA,
        ],
    )
    return k(A, B, src, dst, c, w)


def _dis_of(degT_blk):
    deg = jnp.sum(degT_blk, axis=1, keepdims=True) + 1.0
    return jax.lax.rsqrt(deg)


def _s1_body(degT, x, W, u):
    u[...] = jnp.dot(x[...], W[...],
                     preferred_element_type=jnp.float32) * _dis_of(degT[...])


def _s2_body(degT, a0, a1, up, b, W, u2):
    dis = _dis_of(degT[...])
    h = jnp.maximum((a0[...] + a1[...] + up[...]) * dis + b[...], 0.0)
    u2[...] = jnp.dot(h, W[...], preferred_element_type=jnp.float32) * dis


def _s3_body(degT, a0, a1, up, b, Wa, Wb, A, B):
    dis = _dis_of(degT[...])
    h = (a0[...] + a1[...] + up[...]) * dis + b[...]
    A[...] = jnp.dot(h, Wa[...], preferred_element_type=jnp.float32)
    B[...] = jnp.dot(h, Wb[...], preferred_element_type=jnp.float32)


def _s4_body(e, W, b, c):
    c[...] = jnp.dot(e[...], W[...],
                     preferred_element_type=jnp.float32) + b[...]


def _s5_body(p, b, o):
    o[...] = jnp.sum(p[...], axis=1, keepdims=True) + b[...]


def _node_spec(w=HID):
    return pl.BlockSpec((BN, w), lambda i: (i, 0))


def _full(shape):
    return pl.BlockSpec(shape, lambda i: tuple(0 for _ in shape))


_GRID_N = NPAD // BN


@jax.jit
def _s1(degT, x, W):
    return pl.pallas_call(
        _s1_body,
        grid=(_GRID_N,),
        in_specs=[_node_spec(NW), _node_spec(), _full((HID, HID))],
        out_specs=_node_spec(),
        out_shape=jax.ShapeDtypeStruct((NPAD, HID), jnp.float32),
    )(degT, x, W)


@jax.jit
def _s2(degT, aggp, up, b, W):
    return pl.pallas_call(
        _s2_body,
        grid=(_GRID_N,),
        in_specs=[_node_spec(NW),
                  _node_spec(),
                  pl.BlockSpec((BN, HID), lambda i: (i + _GRID_N, 0)),
                  _node_spec(), _full((1, HID)), _full((HID, HID))],
        out_specs=_node_spec(),
        out_shape=jax.ShapeDtypeStruct((NPAD, HID), jnp.float32),
    )(degT, aggp, aggp, up, b, W)


@jax.jit
def _s3(degT, aggp, up, b, Wa, Wb):
    return pl.pallas_call(
        _s3_body,
        grid=(_GRID_N,),
        in_specs=[_node_spec(NW),
                  _node_spec(),
                  pl.BlockSpec((BN, HID), lambda i: (i + _GRID_N, 0)),
                  _node_spec(), _full((1, HID)),
                  _full((HID, HID)), _full((HID, HID))],
        out_specs=[_node_spec(), _node_spec()],
        out_shape=[jax.ShapeDtypeStruct((NPAD, HID), jnp.float32),
                   jax.ShapeDtypeStruct((NPAD, HID), jnp.float32)],
    )(degT, aggp, aggp, up, b, Wa, Wb)


@jax.jit
def _s4(e, W, b):
    ef = e.shape[1]
    return pl.pallas_call(
        _s4_body,
        grid=(N_EDGES // BE,),
        in_specs=[pl.BlockSpec((BE, ef), lambda i: (i, 0)),
                  _full((ef, HID)), _full((1, HID))],
        out_specs=pl.BlockSpec((BE, HID), lambda i: (i, 0)),
        out_shape=jax.ShapeDtypeStruct((N_EDGES, HID), jnp.float32),
    )(e, W, b)


@jax.jit
def _s5(part, b):
    return pl.pallas_call(
        _s5_body,
        grid=(N_EDGES // BE,),
        in_specs=[pl.BlockSpec((BE, LANES), lambda i: (i, 0)),
                  _full((1, 1))],
        out_specs=pl.BlockSpec((BE, 1), lambda i: (i, 0)),
        out_shape=jax.ShapeDtypeStruct((N_EDGES, 1), jnp.float32),
    )(part, b)


def kernel(x, eIndex, eAttributes, W1, b1, W2, b2, Wm1, bm1, Wm2, bm2):
    src = eIndex[0].astype(jnp.int32)
    dst = eIndex[1].astype(jnp.int32)
    H = W1.shape[1]

    def pad_nodes(m):
        return jnp.concatenate(
            [m, jnp.zeros((NPAD - m.shape[0], m.shape[1]), m.dtype)])

    xp = pad_nodes(x)
    degp = _deg_kernel(dst)
    degT = degp.T

    u = _s1(degT, xp, W1)
    aggp = _agg_kernel(u, src, dst)
    u2 = _s2(degT, aggp, u, b1.reshape(1, H), W2)
    aggp2 = _agg_kernel(u2, src, dst)
    A, B = _s3(degT, aggp2, u2, b2.reshape(1, H), Wm1[:H], Wm1[H:2 * H])

    c = _s4(eAttributes, Wm1[2 * H:], bm1.reshape(1, H))

    part = _edge_stage(A, B, src, dst, c, Wm2[:, 0])
    return _s5(part, bm2.reshape(1, 1))
